# baseline (speedup 1.0000x reference)
import functools

import jax
import jax.numpy as jnp
from jax import lax
from jax.experimental import pallas as pl
from jax.experimental.pallas import tpu as pltpu
from jax.experimental.pallas import tpu_sc as plsc

NC = 2
NS = 16
NW = NC * NS

D = 64
GCHUNK = 128
SUPER = 512
G_PER_SUPER = SUPER // GCHUNK


def _make_gather(B):
    assert B % (NW * SUPER) == 0
    b_per_w = B // NW
    n_super = b_per_w // SUPER
    n_g = b_per_w // GCHUNK
    mesh = plsc.VectorSubcoreMesh(
        core_axis_name="c", subcore_axis_name="s",
        num_cores=NC, num_subcores=NS)

    @functools.partial(
        pl.kernel,
        mesh=mesh,
        compiler_params=pltpu.CompilerParams(use_tc_tiling_on_sc=False),
        out_type=jax.ShapeDtypeStruct((B, D), jnp.float32),
        scratch_types=[
            pltpu.VMEM((n_g, GCHUNK), jnp.int32),
            pltpu.VMEM((2, SUPER, D), jnp.float32),
            pltpu.SemaphoreType.DMA,
            pltpu.SemaphoreType.DMA,
        ],
    )
    def gather_kernel(idx_hbm, table_hbm, out_hbm, idx_v, rows_v, gsem, osem):
        c = lax.axis_index("c")
        s = lax.axis_index("s")
        wid = s * NC + c
        base = wid * b_per_w

        pltpu.sync_copy(idx_hbm.at[wid], idx_v)

        def gather_descr(sidx, j, buf):
            return pltpu.make_async_copy(
                table_hbm.at[idx_v.at[sidx * G_PER_SUPER + j]],
                rows_v.at[buf, pl.ds(j * GCHUNK, GCHUNK)],
                gsem)

        def out_descr(sidx, buf):
            return pltpu.make_async_copy(
                rows_v.at[buf],
                out_hbm.at[pl.ds(base + sidx * SUPER, SUPER)],
                osem)

        def issue_gathers(sidx, buf):
            for j in range(G_PER_SUPER):
                gather_descr(sidx, j, buf).start()

        def wait_gathers(sidx, buf):
            for j in range(G_PER_SUPER):
                gather_descr(sidx, j, buf).wait()

        issue_gathers(0, 0)

        @pl.loop(0, n_super, step=2)
        def _(si):
            for b in range(2):
                sidx = si + b
                wait_gathers(sidx, b)

                @pl.when(sidx > 0)
                def _():
                    out_descr(sidx - 1, 1 - b).wait()

                @pl.when(sidx + 1 < n_super)
                def _():
                    issue_gathers(sidx + 1, 1 - b)

                out_descr(sidx, b).start()

        out_descr(n_super - 1, (n_super - 1) % 2).wait()

    return gather_kernel


def kernel(x, weight):
    B = x.size
    idx = x.reshape(NW, B // (NW * GCHUNK), GCHUNK).astype(jnp.int32)
    out = _make_gather(B)(idx, weight)
    return out.reshape(*x.shape, D)


# speedup vs baseline: 1.0006x; 1.0006x over previous
import functools

import jax
import jax.numpy as jnp
from jax import lax
from jax.experimental import pallas as pl
from jax.experimental.pallas import tpu as pltpu
from jax.experimental.pallas import tpu_sc as plsc

NC = 2
NS = 16
NW = NC * NS

D = 64
CHUNK = 512


def _make_gather(B):
    assert B % (NW * CHUNK) == 0
    b_per_w = B // NW
    n_super = b_per_w // CHUNK
    mesh = plsc.VectorSubcoreMesh(
        core_axis_name="c", subcore_axis_name="s",
        num_cores=NC, num_subcores=NS)

    @functools.partial(
        pl.kernel,
        mesh=mesh,
        compiler_params=pltpu.CompilerParams(use_tc_tiling_on_sc=False),
        out_type=jax.ShapeDtypeStruct((B, D), jnp.float32),
        scratch_types=[
            pltpu.VMEM((b_per_w,), jnp.int32),
            pltpu.VMEM((2, CHUNK, D), jnp.float32),
            pltpu.SemaphoreType.DMA,
            pltpu.SemaphoreType.DMA,
        ],
    )
    def gather_kernel(idx_hbm, table_hbm, out_hbm, idx_v, rows_v, gsem, osem):
        c = lax.axis_index("c")
        s = lax.axis_index("s")
        wid = s * NC + c
        base = wid * b_per_w

        pltpu.sync_copy(idx_hbm.at[pl.ds(base, b_per_w)], idx_v)

        def gather_descr(sidx, buf):
            return pltpu.make_async_copy(
                table_hbm.at[idx_v.at[pl.ds(sidx * CHUNK, CHUNK)]],
                rows_v.at[buf],
                gsem)

        def out_descr(sidx, buf):
            return pltpu.make_async_copy(
                rows_v.at[buf],
                out_hbm.at[pl.ds(base + sidx * CHUNK, CHUNK)],
                osem)

        gather_descr(0, 0).start()

        @pl.loop(0, n_super, step=2)
        def _(si):
            for b in range(2):
                sidx = si + b
                gather_descr(sidx, b).wait()

                @pl.when(sidx > 0)
                def _():
                    out_descr(sidx - 1, 1 - b).wait()

                @pl.when(sidx + 1 < n_super)
                def _():
                    gather_descr(sidx + 1, 1 - b).start()

                out_descr(sidx, b).start()

        out_descr(n_super - 1, (n_super - 1) % 2).wait()

    return gather_kernel


def kernel(x, weight):
    B = x.size
    idx = x.reshape(-1).astype(jnp.int32)
    out = _make_gather(B)(idx, weight)
    return out.reshape(*x.shape, D)
